# Initial kernel scaffold; baseline (speedup 1.0000x reference)
import jax
import jax.numpy as jnp
from jax.experimental import pallas as pl


def kernel(states, priority, edges_mat, edge_index, W_enc, b_enc, W_M, b_M, W_U, b_U, W_nn, b_nn, W_up, b_up, W_t, b_t):
    raise NotImplementedError("write your pallas kernel here")


# speedup vs baseline: 7.3725x; 7.3725x over previous
import functools

import jax
import jax.numpy as jnp
from jax import lax
from jax.experimental import pallas as pl
from jax.experimental.pallas import tpu as pltpu
from jax.experimental.pallas import tpu_sc as plsc

_N = 4096
_E = 131072
_H = 32
_T = 4
_NC = 2
_NS = 16
_NW = _NC * _NS
_NPT = _N // _NW
_CAP = 8192
_CHB = 16384
_GCH = 256
_L = 16

_mesh = plsc.VectorSubcoreMesh(core_axis_name="c", subcore_axis_name="s")
_params = pltpu.CompilerParams(needs_layout_passes=False)


def _wid():
    return lax.axis_index("s") * _NC + lax.axis_index("c")


@functools.partial(
    pl.kernel,
    mesh=_mesh,
    compiler_params=_params,
    out_type=[
        jax.ShapeDtypeStruct((_NW * _CAP,), jnp.int32),
        jax.ShapeDtypeStruct((_NW * _CAP,), jnp.int32),
        jax.ShapeDtypeStruct((_NW * _CAP,), jnp.float32),
        jax.ShapeDtypeStruct((_NW * _L,), jnp.int32),
    ],
    scratch_types=[
        pltpu.VMEM((_CHB,), jnp.int32),
        pltpu.VMEM((_CHB,), jnp.int32),
        pltpu.VMEM((_CAP,), jnp.int32),
        pltpu.VMEM((_CAP,), jnp.int32),
        pltpu.VMEM((_CAP,), jnp.float32),
        pltpu.VMEM((_CAP,), jnp.int32),
        pltpu.VMEM((_CAP,), jnp.int32),
        pltpu.VMEM((_GCH, 128), jnp.float32),
        pltpu.VMEM((_L,), jnp.int32),
        pltpu.SemaphoreType.DMA,
    ],
)
def _prep(src_hbm, dst_hbm, emat_hbm, src_l, dst_l, feat_l, cnt_hbm,
          sbuf, dbuf, src_o, dst_o, feat_o, rowb, colb, grows, cstage, sem):
    w = _wid()
    iota = lax.broadcasted_iota(jnp.int32, (_L,), 0)
    zeros16 = jnp.zeros((_L,), jnp.int32)

    def chunk_body(c, off):
        pltpu.sync_copy(src_hbm.at[pl.ds(c * _CHB, _CHB)], sbuf)
        pltpu.sync_copy(dst_hbm.at[pl.ds(c * _CHB, _CHB)], dbuf)

        def vec_body(i, off):
            s_v = sbuf[pl.ds(i * _L, _L)]
            d_v = dbuf[pl.ds(i * _L, _L)]
            m = (d_v >> 7) == w
            mi = jnp.where(m, jnp.int32(1), jnp.int32(0))
            pos = jnp.minimum(off + plsc.cumsum(mi) - 1, _CAP - 1)
            plsc.store_scatter(src_o, [pos], s_v, mask=m)
            plsc.store_scatter(dst_o, [pos], d_v, mask=m)
            return off + jnp.sum(mi)

        return lax.fori_loop(0, _CHB // _L, vec_body, off)

    cnt = lax.fori_loop(0, _E // _CHB, chunk_body, jnp.int32(0))

    npad = (-cnt) % _GCH
    dump_dst = zeros16 + (w * _NPT + _NPT)

    def pad_body(j, _):
        posp = cnt + j * _L + iota
        mp = posp < cnt + npad
        posp = jnp.minimum(posp, _CAP - 1)
        plsc.store_scatter(src_o, [posp], zeros16, mask=mp)
        plsc.store_scatter(dst_o, [posp], dump_dst, mask=mp)
        return 0

    lax.fori_loop(0, _GCH // _L, pad_body, 0)
    cntp = cnt + npad

    def lin_body(i, _):
        s_v = src_o[pl.ds(i * _L, _L)]
        d_v = dst_o[pl.ds(i * _L, _L)]
        lin = s_v * _N + d_v
        rowb[pl.ds(i * _L, _L)] = lin >> 7
        colb[pl.ds(i * _L, _L)] = lin & 127
        return 0

    lax.fori_loop(0, cntp // _L, lin_body, 0)

    def gch_body(c, _):
        pltpu.async_copy(
            emat_hbm.at[rowb.at[pl.ds(c * _GCH, _GCH)]], grows, sem
        ).wait()

        def ext_body(i, _):
            col = colb[pl.ds(c * _GCH + i * _L, _L)]
            e_loc = i * _L + iota
            feat_o[pl.ds(c * _GCH + i * _L, _L)] = plsc.load_gather(
                grows, [e_loc, col]
            )
            return 0

        lax.fori_loop(0, _GCH // _L, ext_body, 0)
        return 0

    lax.fori_loop(0, cntp // _GCH, gch_body, 0)

    pltpu.sync_copy(src_o, src_l.at[pl.ds(w * _CAP, _CAP)])
    pltpu.sync_copy(dst_o, dst_l.at[pl.ds(w * _CAP, _CAP)])
    pltpu.sync_copy(feat_o, feat_l.at[pl.ds(w * _CAP, _CAP)])
    cstage[pl.ds(0, _L)] = zeros16 + cntp
    pltpu.sync_copy(cstage, cnt_hbm.at[pl.ds(w * _L, _L)])


@functools.partial(
    pl.kernel,
    mesh=_mesh,
    compiler_params=_params,
    out_type=jax.ShapeDtypeStruct((_N * _H,), jnp.float32),
    scratch_types=[
        pltpu.VMEM((_N * _L,), jnp.int32),
        pltpu.VMEM((_CAP,), jnp.int32),
        pltpu.VMEM((_CAP,), jnp.int32),
        pltpu.VMEM((_CAP,), jnp.float32),
        pltpu.VMEM((_L,), jnp.int32),
        pltpu.VMEM(((_NPT + 1) * _H,), jnp.float32),
        pltpu.VMEM((_H,), jnp.float32),
    ],
)
def _msgmax(apack_hbm, src_l, dst_l, feat_l, cnt_hbm, wf_hbm, u_hbm,
            ap, srcb, dstb, featb, cstage, u_loc, wfb):
    w = _wid()
    base = w * _NPT
    pltpu.sync_copy(apack_hbm, ap)
    pltpu.sync_copy(src_l.at[pl.ds(w * _CAP, _CAP)], srcb)
    pltpu.sync_copy(dst_l.at[pl.ds(w * _CAP, _CAP)], dstb)
    pltpu.sync_copy(feat_l.at[pl.ds(w * _CAP, _CAP)], featb)
    pltpu.sync_copy(cnt_hbm.at[pl.ds(w * _L, _L)], cstage)
    pltpu.sync_copy(wf_hbm, wfb)
    n_p = cstage[pl.ds(0, _L)][0]
    wf0 = wfb[pl.ds(0, _L)]
    wf1 = wfb[pl.ds(_L, _L)]
    neg = jnp.full((_L,), -jnp.inf, jnp.float32)

    def fill(i, _):
        u_loc[pl.ds(i * _L, _L)] = neg
        return 0

    lax.fori_loop(0, (_NPT + 1) * _H // _L, fill, 0)

    def g_body(g, _):
        s_v = srcb[pl.ds(g * _L, _L)]
        d_v = dstb[pl.ds(g * _L, _L)]
        f_v = featb[pl.ds(g * _L, _L)]
        for j in range(_L):
            s = s_v[j]
            f = f_v[j]
            off = (d_v[j] - base) * _H
            word = ap[pl.ds(s * _L, _L)]
            a_ev, a_od = plsc.unpack(
                plsc.bitcast(word, jnp.bfloat16),
                format=plsc.PackFormat.INTERLEAVED)
            v0 = a_ev + f * wf0
            v1 = a_od + f * wf1
            u_loc[pl.ds(off, _L)] = jnp.maximum(u_loc[pl.ds(off, _L)], v0)
            u_loc[pl.ds(off + _L, _L)] = jnp.maximum(
                u_loc[pl.ds(off + _L, _L)], v1)
        return 0

    lax.fori_loop(0, n_p // _L, g_body, 0)
    pltpu.sync_copy(u_loc.at[pl.ds(0, _NPT * _H)],
                    u_hbm.at[pl.ds(base * _H, _NPT * _H)])


def _encode_body(x_ref, h_ref, p_ref, wex_ref, weh_ref, wep_ref, be_ref,
                 wms_ref, z_ref, a_ref):
    z = (x_ref[...] * wex_ref[...]
         + jnp.dot(h_ref[...], weh_ref[...],
                   preferred_element_type=jnp.float32)
         + p_ref[...] * wep_ref[...] + be_ref[...])
    z_ref[...] = z
    a_ref[...] = jnp.dot(
        z, wms_ref[...], preferred_element_type=jnp.float32
    ).astype(jnp.bfloat16)


_encode = pl.pallas_call(
    _encode_body,
    out_shape=[jax.ShapeDtypeStruct((_N, _H), jnp.float32),
               jax.ShapeDtypeStruct((_N, _H), jnp.bfloat16)],
)


def _update_body(z_ref, u_ref, wmd_ref, bm_ref, wuz_ref, wuu_ref, bu_ref,
                 wt_ref, bt_ref, wnnh_ref, wnnz_ref, bnn_ref,
                 wuph_ref, wupz_ref, wupn_ref, bup_ref,
                 nh_ref, ns_ref, nne_ref, stop_ref):
    z = z_ref[...]
    u_raw = u_ref[...]
    b = jnp.dot(z, wmd_ref[...], preferred_element_type=jnp.float32)
    u = jnp.where(jnp.isfinite(u_raw), u_raw + b + bm_ref[...], 0.0)
    nh = (jnp.dot(z, wuz_ref[...], preferred_element_type=jnp.float32)
          + jnp.dot(u, wuu_ref[...], preferred_element_type=jnp.float32)
          + bu_ref[...])
    nh_ref[...] = nh
    loc = jnp.dot(nh, wt_ref[...], preferred_element_type=jnp.float32)
    mloc = jnp.dot(jnp.mean(nh, axis=0, keepdims=True), wt_ref[...],
                   preferred_element_type=jnp.float32)
    stop_ref[...] = jax.nn.sigmoid(
        jnp.maximum(jnp.max(loc, axis=0, keepdims=True), mloc) + bt_ref[...])
    nne = (jnp.dot(nh, wnnh_ref[...], preferred_element_type=jnp.float32)
           + jnp.dot(z, wnnz_ref[...], preferred_element_type=jnp.float32)
           + bnn_ref[...])
    nne_ref[...] = nne
    ns_ref[...] = (jnp.dot(nh, wuph_ref[...], preferred_element_type=jnp.float32)
                   + jnp.dot(z, wupz_ref[...], preferred_element_type=jnp.float32)
                   + nne * wupn_ref[...] + bup_ref[...])


_update = pl.pallas_call(
    _update_body,
    out_shape=[jax.ShapeDtypeStruct((_N, _H), jnp.float32),
               jax.ShapeDtypeStruct((_N, 1), jnp.float32),
               jax.ShapeDtypeStruct((_N, 1), jnp.float32),
               jax.ShapeDtypeStruct((1, 1), jnp.float32)],
)

_SIG = list(range(0, _H, 2)) + list(range(1, _H, 2))


def kernel(states, priority, edges_mat, edge_index,
           W_enc, b_enc, W_M, b_M, W_U, b_U,
           W_nn, b_nn, W_up, b_up, W_t, b_t):
    src = edge_index[0]
    dst = edge_index[1]
    emat128 = edges_mat.reshape(_N * _N // 128, 128)
    src_l, dst_l, feat_l, cnt = _prep(src, dst, emat128)

    sig = jnp.asarray(_SIG, jnp.int32)
    wex = W_enc[0:1]
    weh = W_enc[1:33]
    wep = W_enc[33:34]
    be = b_enc[None]
    wms = W_M[:32]
    wmd = W_M[32:64][:, sig]
    wf = W_M[64][sig]
    bm = b_M[sig][None]
    wuz = W_U[:32]
    wuu = W_U[32:][sig, :]
    bu = b_U[None]
    wnnh = W_nn[:32]
    wnnz = W_nn[32:]
    bnn = b_nn[None]
    wuph = W_up[:32]
    wupz = W_up[32:64]
    wupn = W_up[64:65]
    bup = b_up[None]
    bt = b_t[None]

    hidden = jnp.zeros((_N, _H), jnp.float32)
    x = states[0][:, None]
    prio = priority[:, None]
    pred_all = [x]
    pred_stop = [jnp.zeros((1, 1), jnp.float32)]
    pred_next = []
    for _ in range(_T - 1):
        z, a_bf = _encode(x, hidden, prio, wex, weh, wep, be, wms)
        apack = lax.bitcast_convert_type(
            a_bf.reshape(_N, _L, 2), jnp.int32).reshape(_N * _L)
        u_raw = _msgmax(apack, src_l, dst_l, feat_l, cnt, wf).reshape(_N, _H)
        nh, ns, nne, stop = _update(z, u_raw, wmd, bm, wuz, wuu, bu,
                                    W_t, bt, wnnh, wnnz, bnn,
                                    wuph, wupz, wupn, bup)
        hidden = nh
        x = ns
        pred_all.append(ns)
        pred_stop.append(stop)
        pred_next.append(nne)

    preds = jnp.stack(pred_all, axis=1).reshape(_T, _N)
    preds_stop = jnp.stack(pred_stop, axis=1)
    preds_nextnode = jnp.stack(pred_next, axis=1)
    return (preds, preds_stop, preds_nextnode)
